# Initial kernel scaffold; baseline (speedup 1.0000x reference)
import jax
import jax.numpy as jnp
from jax.experimental import pallas as pl


def kernel(x, center, weight):
    raise NotImplementedError("write your pallas kernel here")


# speedup vs baseline: 9.9323x; 9.9323x over previous
import functools

import jax
import jax.numpy as jnp
from jax.experimental import pallas as pl

B, M, K, D = 8192, 4, 1024, 64
BLK = 512


def _stage_body(first, x_ref, xpm_ref, ct_ref, idx_ref, xpm_out_ref, xr_ref,
                rx_ref, osum_ref, ssum_ref, msum_ref):
    i = pl.program_id(0)
    ct = ct_ref[...]
    cn = jnp.sum(ct * ct, axis=0, keepdims=True)
    if first:
        rx = x_ref[...]
        xpm_prev = None
    else:
        xpm_prev = xpm_ref[...]
        rx = x_ref[...] - xpm_prev

    dot = jax.lax.dot_general(
        rx, ct, (((1,), (0,)), ((), ())),
        precision=jax.lax.Precision.HIGHEST,
        preferred_element_type=jnp.float32)
    xn = jnp.sum(rx * rx, axis=1, keepdims=True)
    sq = jnp.maximum((xn + cn) - 2.0 * dot, 0.0)
    att = -jnp.sqrt(sq)

    m = jnp.max(att, axis=1, keepdims=True)
    e = jnp.exp(att - m)
    s = jnp.sum(e, axis=1, keepdims=True)
    soft = e / s

    smax = jnp.max(soft, axis=1, keepdims=True)
    iota = jax.lax.broadcasted_iota(jnp.int32, soft.shape, 1)
    idx = jnp.min(jnp.where(soft == smax, iota, K), axis=1,
                  keepdims=True)

    cbt = ct.astype(jnp.bfloat16)
    oh = (iota == idx).astype(jnp.bfloat16)
    xp = jax.lax.dot_general(
        oh, cbt, (((1,), (1,)), ((), ())),
        preferred_element_type=jnp.float32)

    softc = jax.lax.dot_general(
        soft.astype(jnp.bfloat16), cbt, (((1,), (1,)), ((), ())),
        preferred_element_type=jnp.float32)

    sm = smax[:, 0]
    r1 = (jnp.float32(1.0) - sm) + sm
    match_part = jnp.sum(jnp.sum(soft * soft, axis=1) - sm * sm
                         + (r1 - sm) * (r1 - sm))
    sout_part = jnp.sum((softc - rx) ** 2)
    out_part = jnp.sum((xp - rx) ** 2)

    idx_ref[...] = idx
    xr_ref[...] = xp
    rx_ref[...] = rx
    xpm_out_ref[...] = xp if first else xpm_prev + xp

    @pl.when(i == 0)
    def _init():
        osum_ref[...] = out_part[None, None]
        ssum_ref[...] = sout_part[None, None]
        msum_ref[...] = match_part[None, None]

    @pl.when(i != 0)
    def _acc():
        osum_ref[...] += out_part[None, None]
        ssum_ref[...] += sout_part[None, None]
        msum_ref[...] += match_part[None, None]


def _stage_call(first, x, xpm, ct):
    grid = (B // BLK,)
    row_spec = pl.BlockSpec((BLK, D), lambda i: (i, 0))
    scal_spec = pl.BlockSpec((1, 1), lambda i: (0, 0))
    in_specs = [row_spec]
    args = [x]
    if not first:
        in_specs.append(row_spec)
        args.append(xpm)
    else:
        in_specs.append(row_spec)
        args.append(x)
    in_specs.append(pl.BlockSpec((D, K), lambda i: (0, 0)))
    args.append(ct)
    out_shape = [
        jax.ShapeDtypeStruct((B, 1), jnp.int32),
        jax.ShapeDtypeStruct((B, D), jnp.float32),
        jax.ShapeDtypeStruct((B, D), jnp.float32),
        jax.ShapeDtypeStruct((B, D), jnp.float32),
        jax.ShapeDtypeStruct((1, 1), jnp.float32),
        jax.ShapeDtypeStruct((1, 1), jnp.float32),
        jax.ShapeDtypeStruct((1, 1), jnp.float32),
    ]
    out_specs = [
        pl.BlockSpec((BLK, 1), lambda i: (i, 0)),
        row_spec, row_spec, row_spec,
        scal_spec, scal_spec, scal_spec,
    ]
    return pl.pallas_call(
        functools.partial(_stage_body, first),
        grid=grid, in_specs=in_specs, out_specs=out_specs,
        out_shape=out_shape)(*args)


def _combine_body(os_ref, ss_ref, ms_ref, w_ref, out_ref):
    scale = jnp.float32(1.0 / (B * D))
    mscale = jnp.float32(1.0 / (B * K))
    w = w_ref[0, :]
    lquanH = jnp.sum(w * (os_ref[0, :] * scale))
    lquan = jnp.sum(w * (ss_ref[0, :] * scale))
    lmatch = jnp.sum(w * (ms_ref[0, :] * mscale))
    out_ref[...] = (lquanH + 0.1 * lmatch + lquan)[None, None]


def kernel(x, center, weight):
    idxs, xrs, rxs = [], [], []
    osums, ssums, msums = [], [], []
    xpm = None
    for j in range(M):
        idx, xpm, xr, rx, osum, ssum, msum = _stage_call(
            j == 0, x, xpm, center[j].T)
        idxs.append(idx)
        xrs.append(xr)
        rxs.append(rx)
        osums.append(osum)
        ssums.append(ssum)
        msums.append(msum)

    os_ = jnp.concatenate(osums, axis=1)
    ss_ = jnp.concatenate(ssums, axis=1)
    ms_ = jnp.concatenate(msums, axis=1)
    out = pl.pallas_call(
        _combine_body,
        out_shape=jax.ShapeDtypeStruct((1, 1), jnp.float32),
    )(os_, ss_, ms_, weight)

    X_r_matrix = jnp.stack(xrs, axis=1)
    X_p_matrix = jnp.stack(rxs, axis=1)
    codes = jnp.concatenate(idxs, axis=1)
    codebooks = center.reshape(M * K, D)
    return (X_r_matrix, X_p_matrix, xpm, xpm, codebooks, codes, out)
